# baseline (speedup 1.0000x reference)
import functools

import jax
import jax.numpy as jnp
from jax import lax
from jax.experimental import pallas as pl
from jax.experimental.pallas import tpu as pltpu
from jax.experimental.pallas import tpu_sc as plsc

_NC = 2
_NS = 16
_NW = _NC * _NS
_CB = 256
_TP = 129


def _emb_kernel(n_blocks_per_w, blocks_per_s,
                tids_hbm, table_hbm, out_hbm,
                idx0, idx1, g0, g1, t0, t1, sg0, sg1, so0, so1):
    wid = lax.axis_index("s") * _NC + lax.axis_index("c")
    q0 = wid * n_blocks_per_w
    idxv = (idx0, idx1)
    G = (g0, g1)
    GT = (t0, t1)
    sg = (sg0, sg1)
    so = (so0, so1)

    iota = lax.iota(jnp.int32, 16)

    def load_idx_and_gather(blk, p):
        s = blk // blocks_per_s
        b0 = (blk - s * blocks_per_s) * _CB
        pltpu.sync_copy(tids_hbm.at[s, pl.ds(b0, _CB)], idxv[p])
        pltpu.make_async_copy(table_hbm.at[idxv[p]], G[p], sg[p]).start()

    def wait_gather(p):
        pltpu.make_async_copy(table_hbm.at[idxv[p]], G[p], sg[p]).wait()

    def out_copy(blk, p, j, chi):
        s = blk // blocks_per_s
        bhi = (blk - s * blocks_per_s) * 2 + j
        return pltpu.make_async_copy(
            GT[p].at[pl.ds(j * 64 + chi * 8, 8), pl.ds(0, 128)],
            out_hbm.at[s, chi, bhi], so[p])

    def transpose_block(p):
        for t in range(_CB):
            j = t // 128
            tl = t - j * 128
            col = jnp.full((16,), tl, jnp.int32)
            for k in range(4):
                vals = G[p][t, pl.ds(16 * k, 16)]
                rows = iota + (j * 64 + 16 * k)
                plsc.store_scatter(GT[p], [rows, col], vals)

    load_idx_and_gather(q0, 0)

    def body(i, carry):
        for p in (0, 1):
            blk = q0 + 2 * i + p
            nxt = 2 * i + p + 1

            @pl.when(nxt < n_blocks_per_w)
            def _():
                load_idx_and_gather(blk + 1, 1 - p)

            wait_gather(p)

            @pl.when(2 * i + p >= 2)
            def _():
                for j in (0, 1):
                    for chi in range(8):
                        out_copy(blk - 2, p, j, chi).wait()

            transpose_block(p)
            for j in (0, 1):
                for chi in range(8):
                    out_copy(blk, p, j, chi).start()
        return carry

    lax.fori_loop(0, n_blocks_per_w // 2, body, 0)

    for p in (0, 1):
        blk = q0 + n_blocks_per_w - 2 + p
        for j in (0, 1):
            for chi in range(8):
                out_copy(blk, p, j, chi).wait()


def kernel(token_ids, weight):
    bsz, seq = token_ids.shape
    nv, d = weight.shape
    tids_lin = token_ids.T.astype(jnp.int32)

    blocks_per_s = bsz // _CB
    n_blocks = seq * blocks_per_s
    n_blocks_per_w = n_blocks // _NW

    mesh = plsc.VectorSubcoreMesh(core_axis_name="c", subcore_axis_name="s")
    k = functools.partial(
        pl.kernel,
        mesh=mesh,
        out_type=jax.ShapeDtypeStruct((seq, 8, bsz // 128, 8, 128),
                                      jnp.float32),
        scratch_types=[
            pltpu.VMEM((_CB,), jnp.int32),
            pltpu.VMEM((_CB,), jnp.int32),
            pltpu.VMEM((_CB, d), jnp.float32),
            pltpu.VMEM((_CB, d), jnp.float32),
            pltpu.VMEM((128, _TP), jnp.float32),
            pltpu.VMEM((128, _TP), jnp.float32),
            pltpu.SemaphoreType.DMA,
            pltpu.SemaphoreType.DMA,
            pltpu.SemaphoreType.DMA,
            pltpu.SemaphoreType.DMA,
        ],
        compiler_params=pltpu.CompilerParams(use_tc_tiling_on_sc=False,
                                             needs_layout_passes=False),
    )(functools.partial(_emb_kernel, n_blocks_per_w, blocks_per_s))

    out = k(tids_lin, weight)
    return jnp.transpose(out, (2, 4, 0, 1, 3)).reshape(bsz, seq, d)


# speedup vs baseline: 1.0178x; 1.0178x over previous
import functools

import jax
import jax.numpy as jnp
from jax import lax
from jax.experimental import pallas as pl
from jax.experimental.pallas import tpu as pltpu
from jax.experimental.pallas import tpu_sc as plsc

_NC = 2
_NS = 16
_NW = _NC * _NS
_CB = 256
_TP = 129


def _emb_kernel(n_blocks_per_w, blocks_per_s,
                tids_hbm, table_hbm, out_hbm,
                idx0, idx1, g0, g1, t0, t1, sg0, sg1, so0, so1):
    wid = lax.axis_index("s") * _NC + lax.axis_index("c")
    q0 = wid * n_blocks_per_w
    idxv = (idx0, idx1)
    G = (g0, g1)
    GT = (t0, t1)
    sg = (sg0, sg1)
    so = (so0, so1)

    iota = lax.iota(jnp.int32, 16)

    def load_idx_and_gather(blk, p):
        s = blk // blocks_per_s
        b0 = (blk - s * blocks_per_s) * _CB
        pltpu.sync_copy(tids_hbm.at[s, pl.ds(b0, _CB)], idxv[p])
        pltpu.make_async_copy(table_hbm.at[idxv[p]], G[p], sg[p]).start()

    def wait_gather(p):
        pltpu.make_async_copy(table_hbm.at[idxv[p]], G[p], sg[p]).wait()

    def out_copy(blk, p, j, chi):
        s = blk // blocks_per_s
        bhi = (blk - s * blocks_per_s) * 2 + j
        return pltpu.make_async_copy(
            GT[p].at[pl.ds(j * 64 + chi * 8, 8), pl.ds(0, 128)],
            out_hbm.at[s, chi, bhi], so[p])

    rows_jk = {(j, k): iota + (j * 64 + 16 * k)
               for j in (0, 1) for k in range(4)}

    def transpose_block(p):
        prev = None
        for t in range(_CB):
            j = t // 128
            tl = t - j * 128
            col = jnp.full((16,), tl, jnp.int32)
            cur = (j, col, [G[p][t, pl.ds(16 * k, 16)] for k in range(4)])
            if prev is not None:
                pj, pcol, pvals = prev
                for k in range(4):
                    plsc.store_scatter(GT[p], [rows_jk[(pj, k)], pcol],
                                       pvals[k])
            prev = cur
        pj, pcol, pvals = prev
        for k in range(4):
            plsc.store_scatter(GT[p], [rows_jk[(pj, k)], pcol], pvals[k])

    load_idx_and_gather(q0, 0)

    def body(i, carry):
        for p in (0, 1):
            blk = q0 + 2 * i + p
            nxt = 2 * i + p + 1

            @pl.when(nxt < n_blocks_per_w)
            def _():
                load_idx_and_gather(blk + 1, 1 - p)

            wait_gather(p)

            @pl.when(2 * i + p >= 2)
            def _():
                for j in (0, 1):
                    for chi in range(8):
                        out_copy(blk - 2, p, j, chi).wait()

            transpose_block(p)
            for j in (0, 1):
                for chi in range(8):
                    out_copy(blk, p, j, chi).start()
        return carry

    lax.fori_loop(0, n_blocks_per_w // 2, body, 0)

    for p in (0, 1):
        blk = q0 + n_blocks_per_w - 2 + p
        for j in (0, 1):
            for chi in range(8):
                out_copy(blk, p, j, chi).wait()


def kernel(token_ids, weight):
    bsz, seq = token_ids.shape
    nv, d = weight.shape
    tids_lin = token_ids.T.astype(jnp.int32)

    blocks_per_s = bsz // _CB
    n_blocks = seq * blocks_per_s
    n_blocks_per_w = n_blocks // _NW

    mesh = plsc.VectorSubcoreMesh(core_axis_name="c", subcore_axis_name="s")
    k = functools.partial(
        pl.kernel,
        mesh=mesh,
        out_type=jax.ShapeDtypeStruct((seq, 8, bsz // 128, 8, 128),
                                      jnp.float32),
        scratch_types=[
            pltpu.VMEM((_CB,), jnp.int32),
            pltpu.VMEM((_CB,), jnp.int32),
            pltpu.VMEM((_CB, d), jnp.float32),
            pltpu.VMEM((_CB, d), jnp.float32),
            pltpu.VMEM((128, _TP), jnp.float32),
            pltpu.VMEM((128, _TP), jnp.float32),
            pltpu.SemaphoreType.DMA,
            pltpu.SemaphoreType.DMA,
            pltpu.SemaphoreType.DMA,
            pltpu.SemaphoreType.DMA,
        ],
        compiler_params=pltpu.CompilerParams(use_tc_tiling_on_sc=False,
                                             needs_layout_passes=False),
    )(functools.partial(_emb_kernel, n_blocks_per_w, blocks_per_s))

    out = k(tids_lin, weight)
    return jnp.transpose(out, (2, 4, 0, 1, 3)).reshape(bsz, seq, d)


# speedup vs baseline: 1.3429x; 1.3194x over previous
import functools

import jax
import jax.numpy as jnp
from jax import lax
from jax.experimental import pallas as pl
from jax.experimental.pallas import tpu as pltpu
from jax.experimental.pallas import tpu_sc as plsc

_NC = 2
_NS = 16
_NW = _NC * _NS
_CHUNK = 512
_DP = 128


def _gather_kernel(n_chunks, b_per_w,
                   idx_hbm, table_hbm, out_hbm,
                   idx_all, rows0, rows1, sg0, sg1, ss0, ss1):
    wid = lax.axis_index("s") * _NC + lax.axis_index("c")
    base = wid * b_per_w
    rows = (rows0, rows1)
    sg = (sg0, sg1)
    ss = (ss0, ss1)

    pltpu.sync_copy(idx_hbm.at[pl.ds(base, b_per_w)], idx_all)

    def gather_copy(c, b):
        return pltpu.make_async_copy(
            table_hbm.at[idx_all.at[pl.ds(c * _CHUNK, _CHUNK)]], rows[b], sg[b])

    def store_copy(c, b):
        return pltpu.make_async_copy(
            rows[b],
            out_hbm.at[pl.ds(base + c * _CHUNK, _CHUNK), pl.ds(0, 64)],
            ss[b])

    gather_copy(0, 0).start()
    gather_copy(1, 1).start()

    def body(g, carry):
        c0 = 2 * g
        for b in (0, 1):
            gather_copy(c0 + b, b).wait()
            store_copy(c0 + b, b).start()
        for b in (0, 1):
            store_copy(c0 + b, b).wait()
            gather_copy(c0 + 2 + b, b).start()
        return carry

    n_groups = n_chunks // 2
    lax.fori_loop(0, n_groups - 1, body, 0)

    c0 = n_chunks - 2
    for b in (0, 1):
        gather_copy(c0 + b, b).wait()
        store_copy(c0 + b, b).start()
    for b in (0, 1):
        store_copy(c0 + b, b).wait()


def kernel(token_ids, weight):
    bsz, seq = token_ids.shape
    nv, d = weight.shape
    n = bsz * seq
    idx_flat = token_ids.reshape(n).astype(jnp.int32)

    b_per_w = n // _NW
    n_chunks = b_per_w // _CHUNK

    mesh = plsc.VectorSubcoreMesh(core_axis_name="c", subcore_axis_name="s")
    k = functools.partial(
        pl.kernel,
        mesh=mesh,
        out_type=jax.ShapeDtypeStruct((n, _DP), jnp.float32),
        scratch_types=[
            pltpu.VMEM((b_per_w,), jnp.int32),
            pltpu.VMEM((_CHUNK, d), jnp.float32),
            pltpu.VMEM((_CHUNK, d), jnp.float32),
            pltpu.SemaphoreType.DMA,
            pltpu.SemaphoreType.DMA,
            pltpu.SemaphoreType.DMA,
            pltpu.SemaphoreType.DMA,
        ],
        compiler_params=pltpu.CompilerParams(use_tc_tiling_on_sc=False,
                                             skip_device_barrier=True),
    )(functools.partial(_gather_kernel, n_chunks, b_per_w))

    out = k(idx_flat, weight)
    return out[:, :d].reshape(bsz, seq, d)


# speedup vs baseline: 1.3484x; 1.0041x over previous
import functools

import jax
import jax.numpy as jnp
from jax import lax
from jax.experimental import pallas as pl
from jax.experimental.pallas import tpu as pltpu
from jax.experimental.pallas import tpu_sc as plsc

_NC = 2
_NS = 16
_NW = _NC * _NS
_CHUNK = 512
_DP = 128


def _gather_kernel(n_chunks, b_per_w,
                   idx_hbm, table_hbm, out_hbm,
                   idx_all, rows0, rows1, sg0, sg1, ss0, ss1):
    wid = lax.axis_index("s") * _NC + lax.axis_index("c")
    base = wid * b_per_w
    rows = (rows0, rows1)
    sg = (sg0, sg1)
    ss = (ss0, ss1)

    pltpu.sync_copy(idx_hbm.at[pl.ds(base, b_per_w)], idx_all)

    def gather_copy(c, b):
        return pltpu.make_async_copy(
            table_hbm.at[idx_all.at[pl.ds(c * _CHUNK, _CHUNK)]], rows[b], sg[b])

    def store_copy(c, b):
        return pltpu.make_async_copy(
            rows[b],
            out_hbm.at[pl.ds(base + c * _CHUNK, _CHUNK), pl.ds(0, 64)],
            ss[b])

    gather_copy(0, 0).start()
    gather_copy(1, 1).start()

    def body(g, carry):
        c0 = 2 * g
        for b in (0, 1):
            gather_copy(c0 + b, b).wait()
            store_copy(c0 + b, b).start()
        for b in (0, 1):
            store_copy(c0 + b, b).wait()
            gather_copy(c0 + 2 + b, b).start()
        return carry

    n_groups = n_chunks // 2
    lax.fori_loop(0, n_groups - 1, body, 0)

    c0 = n_chunks - 2
    for b in (0, 1):
        gather_copy(c0 + b, b).wait()
        store_copy(c0 + b, b).start()
    for b in (0, 1):
        store_copy(c0 + b, b).wait()


def kernel(token_ids, weight):
    bsz, seq = token_ids.shape
    nv, d = weight.shape
    n = bsz * seq
    idx_flat = token_ids.reshape(n).astype(jnp.int32)

    b_per_w = n // _NW
    n_chunks = b_per_w // _CHUNK

    mesh = plsc.VectorSubcoreMesh(core_axis_name="c", subcore_axis_name="s")
    k = functools.partial(
        pl.kernel,
        mesh=mesh,
        out_type=jax.ShapeDtypeStruct((n, _DP), jnp.float32),
        scratch_types=[
            pltpu.VMEM((b_per_w,), jnp.int32),
            pltpu.VMEM((_CHUNK, d), jnp.float32),
            pltpu.VMEM((_CHUNK, d), jnp.float32),
            pltpu.SemaphoreType.DMA,
            pltpu.SemaphoreType.DMA,
            pltpu.SemaphoreType.DMA,
            pltpu.SemaphoreType.DMA,
        ],
        compiler_params=pltpu.CompilerParams(use_tc_tiling_on_sc=False),
    )(functools.partial(_gather_kernel, n_chunks, b_per_w))

    out = k(idx_flat, weight)
    return out[:, :d].reshape(bsz, seq, d)
